# baseline (speedup 1.0000x reference)
import functools

import jax
import jax.numpy as jnp
from jax import lax
from jax.experimental import pallas as pl
from jax.experimental.pallas import tpu as pltpu
from jax.experimental.pallas import tpu_sc as plsc

_NC = 2
_NS = 16
_NW = _NC * _NS
_L = 16
_BCH = 128


def kernel(inputs, table):
    B, F = inputs.shape
    V, D = table.shape
    bpw = B // _NW
    steps = F * (bpw // _BCH)
    n_ach = V // 128
    tail_v0 = n_ach * 128
    tail_w = V - tail_v0

    idx_t = inputs.T.astype(jnp.int32)
    table_t = table.T
    tail_t = jnp.pad(table[tail_v0:].T, ((0, 128 - D), (0, 128 - tail_w)))
    mesh = plsc.VectorSubcoreMesh(core_axis_name="c", subcore_axis_name="s")

    @functools.partial(
        pl.kernel,
        mesh=mesh,
        out_type=jax.ShapeDtypeStruct((F, D, B), jnp.float32),
        compiler_params=pltpu.CompilerParams(
            use_tc_tiling_on_sc=True,
            needs_layout_passes=False,
        ),
        scratch_types=(
            [
                pltpu.HBM((V // 2, 2 * D), jnp.float32),
                pltpu.VMEM((2, D, 128), jnp.float32),
                pltpu.VMEM((2, D, 2 * D), jnp.float32),
                pltpu.VMEM((F, bpw), jnp.int32),
                pltpu.VMEM((F, bpw), jnp.int32),
                pltpu.VMEM((2, _BCH, 2 * D), jnp.float32),
                pltpu.VMEM((2, D, _BCH), jnp.float32),
            ]
            + [pltpu.SemaphoreType.DMA] * 8
            + [pltpu.SemaphoreType.REGULAR]
        ),
    )
    def run(tab_hbm, idx_hbm, tail_hbm, out_hbm, scr_hbm,
            ain_v, apk_v, idx_v, kidx_v, g_v, t_v, *sems):
        isem = sems[0:2]
        oasem = sems[2:4]
        gsem = sems[4:6]
        obsem = sems[6:8]
        xsem = sems[8]
        cid = lax.axis_index("c")
        sid = lax.axis_index("s")
        wid = sid * _NC + cid
        iot = lax.iota(jnp.int32, _L)

        nc_w = (n_ach - wid + _NW - 1) // _NW

        def a_v0(i):
            return (wid + i * _NW) * 128

        def a_start_in(i, slot):
            pltpu.async_copy(
                tab_hbm.at[:, pl.ds(a_v0(i), 128)], ain_v.at[slot], isem[slot]
            )

        def a_wait_in(i, slot):
            pltpu.make_async_copy(
                tab_hbm.at[:, pl.ds(a_v0(i), 128)], ain_v.at[slot], isem[slot]
            ).wait()

        def a_dst(i):
            return scr_hbm.at[pl.ds(a_v0(i) // 2, D)]

        def a_transpose(slot):
            row0 = lax.shift_right_logical(iot, 1)
            hvec = jnp.bitwise_and(iot, 1) * D

            @plsc.parallel_loop(0, _L, unroll=4)
            def arot(r):
                rot = jnp.bitwise_and(iot + r, _L - 1)
                for jb2 in range(128 // _L):
                    colvec = iot + jb2 * _L
                    rowvec = row0 + (jb2 * (_L // 2))
                    vals = [plsc.load_gather(ain_v.at[slot],
                                             [rot + (db * _L), colvec])
                            for db in range(D // _L)]
                    for db in range(D // _L):
                        plsc.store_scatter(apk_v.at[slot],
                                           [rowvec, hvec + rot + (db * _L)],
                                           vals[db])

        def a_body(i, slot):
            @pl.when(i + 1 < nc_w)
            def _():
                a_start_in(i + 1, 1 - slot)

            a_wait_in(i, slot)

            @pl.when(i >= 2)
            def _():
                pltpu.make_async_copy(apk_v.at[slot], a_dst(i - 2),
                                      oasem[slot]).wait()

            a_transpose(slot)
            pltpu.async_copy(apk_v.at[slot], a_dst(i), oasem[slot])

        a_start_in(0, 0)

        def a_pair(p, carry):
            a_body(2 * p, 0)
            a_body(2 * p + 1, 1)
            return carry

        lax.fori_loop(0, nc_w // 2, a_pair, 0)

        @pl.when(lax.rem(nc_w, 2) == 1)
        def _():
            a_body(nc_w - 1, 0)

        @pl.when(lax.rem(nc_w, 2) == 1)
        def _():
            pltpu.make_async_copy(apk_v.at[1], a_dst(nc_w - 2), oasem[1]).wait()
            pltpu.make_async_copy(apk_v.at[0], a_dst(nc_w - 1), oasem[0]).wait()

        @pl.when(lax.rem(nc_w, 2) == 0)
        def _():
            pltpu.make_async_copy(apk_v.at[0], a_dst(nc_w - 2), oasem[0]).wait()
            pltpu.make_async_copy(apk_v.at[1], a_dst(nc_w - 1), oasem[1]).wait()

        @pl.when(wid == 0)
        def _():
            pltpu.sync_copy(tail_hbm, g_v.at[0])

            def trot(r, carry):
                rot = jnp.bitwise_and(iot + r, _L - 1)

                def tblk(jb2, carry2):
                    colvec = iot + jb2 * _L
                    rowvec = (jb2 * (_L // 2)) + lax.shift_right_logical(iot, 1)
                    hvec = jnp.bitwise_and(iot, 1) * D

                    def tdb(db, carry3):
                        dvec = rot + (db * _L)
                        vals = plsc.load_gather(g_v.at[0], [dvec, colvec])
                        plsc.store_scatter(apk_v.at[0],
                                           [rowvec, hvec + dvec], vals)
                        return carry3
                    return lax.fori_loop(0, D // _L, tdb, carry2)
                return lax.fori_loop(0, tail_w // _L, tblk, carry)

            lax.fori_loop(0, _L, trot, 0)

            pltpu.sync_copy(apk_v.at[0, pl.ds(0, tail_w // 2)],
                            scr_hbm.at[pl.ds(tail_v0 // 2, tail_w // 2)])

        plsc.subcore_barrier()

        @pl.when(sid == 0)
        def _():
            pl.semaphore_signal(xsem, 1, core_index=1 - cid)
            pl.semaphore_wait(xsem, 1)

        plsc.subcore_barrier()

        b0 = wid * bpw
        pltpu.sync_copy(idx_hbm.at[:, pl.ds(b0, bpw)], idx_v)

        def halve(f, carry):
            for jb in range(bpw // _L):
                v16 = idx_v[f, pl.ds(jb * _L, _L)]
                kidx_v[f, pl.ds(jb * _L, _L)] = lax.shift_right_logical(v16, 1)
            return carry

        lax.fori_loop(0, F, halve, 0)

        def b_src(s):
            f = s // (bpw // _BCH)
            c = lax.rem(s, bpw // _BCH)
            return scr_hbm.at[kidx_v.at[f, pl.ds(c * _BCH, _BCH)]]

        def b_start_gather(s, slot):
            pltpu.async_copy(b_src(s), g_v.at[slot], gsem[slot])

        def b_wait_gather(s, slot):
            pltpu.make_async_copy(b_src(s), g_v.at[slot], gsem[slot]).wait()

        def b_dst(s):
            f = s // (bpw // _BCH)
            c = lax.rem(s, bpw // _BCH)
            return out_hbm.at[f, :, pl.ds(b0 + c * _BCH, _BCH)]

        def b_transpose(s, slot):
            f = s // (bpw // _BCH)
            c = lax.rem(s, bpw // _BCH)

            @plsc.parallel_loop(0, _L, unroll=4)
            def brot(r):
                rot = jnp.bitwise_and(iot + r, _L - 1)
                for jb in range(_BCH // _L):
                    j0 = jb * _L
                    jvec = iot + j0
                    v16 = idx_v[f, pl.ds(c * _BCH + j0, _L)]
                    hv = lax.shift_left(jnp.bitwise_and(v16, 1), 6)
                    vals = [plsc.load_gather(g_v.at[slot],
                                             [jvec, rot + (db * _L) + hv])
                            for db in range(D // _L)]
                    for db in range(D // _L):
                        plsc.store_scatter(t_v.at[slot],
                                           [rot + (db * _L), jvec], vals[db])

        def b_body(s, slot):
            @pl.when(s + 1 < steps)
            def _():
                b_start_gather(s + 1, 1 - slot)

            b_wait_gather(s, slot)

            @pl.when(s >= 2)
            def _():
                pltpu.make_async_copy(t_v.at[slot], b_dst(s - 2),
                                      obsem[slot]).wait()

            b_transpose(s, slot)
            pltpu.async_copy(t_v.at[slot], b_dst(s), obsem[slot])

        b_start_gather(0, 0)

        def b_pair(p, carry):
            b_body(2 * p, 0)
            b_body(2 * p + 1, 1)
            return carry

        lax.fori_loop(0, steps // 2, b_pair, 0)

        pltpu.make_async_copy(t_v.at[0], b_dst(steps - 2), obsem[0]).wait()
        pltpu.make_async_copy(t_v.at[1], b_dst(steps - 1), obsem[1]).wait()

    out_t = run(table_t, idx_t, tail_t)
    return jnp.transpose(out_t, (2, 0, 1))


# speedup vs baseline: 1.0345x; 1.0345x over previous
import functools

import jax
import jax.numpy as jnp
from jax import lax
from jax.experimental import pallas as pl
from jax.experimental.pallas import tpu as pltpu
from jax.experimental.pallas import tpu_sc as plsc

_NC = 2
_NS = 16
_NW = _NC * _NS
_L = 16
_BCH = 128


def kernel(inputs, table):
    B, F = inputs.shape
    V, D = table.shape
    bpw = B // _NW
    steps = F * (bpw // _BCH)
    n_ach = V // 128
    tail_v0 = n_ach * 128
    tail_w = V - tail_v0

    idx_t = inputs.T.astype(jnp.int32)
    table_t = table.T
    tail_t = jnp.pad(table[tail_v0:].T, ((0, 128 - D), (0, 128 - tail_w)))
    mesh = plsc.VectorSubcoreMesh(core_axis_name="c", subcore_axis_name="s")

    @functools.partial(
        pl.kernel,
        mesh=mesh,
        out_type=jax.ShapeDtypeStruct((F, D, B), jnp.float32),
        compiler_params=pltpu.CompilerParams(
            use_tc_tiling_on_sc=True,
            needs_layout_passes=False,
        ),
        scratch_types=(
            [
                pltpu.HBM((V // 2, 2 * D), jnp.float32),
                pltpu.VMEM((2, D, 128), jnp.float32),
                pltpu.VMEM((2, D, 2 * D), jnp.float32),
                pltpu.VMEM((F, bpw), jnp.int32),
                pltpu.VMEM((F, bpw), jnp.int32),
                pltpu.VMEM((2, _BCH, 2 * D), jnp.float32),
                pltpu.VMEM((2, D, _BCH), jnp.float32),
            ]
            + [pltpu.SemaphoreType.DMA] * 8
            + [pltpu.SemaphoreType.REGULAR]
        ),
    )
    def run(tab_hbm, idx_hbm, tail_hbm, out_hbm, scr_hbm,
            ain_v, apk_v, idx_v, kidx_v, g_v, t_v, *sems):
        isem = sems[0:2]
        oasem = sems[2:4]
        gsem = sems[4:6]
        obsem = sems[6:8]
        xsem = sems[8]
        cid = lax.axis_index("c")
        sid = lax.axis_index("s")
        wid = sid * _NC + cid
        iot = lax.iota(jnp.int32, _L)

        nc_w = (n_ach - wid + _NW - 1) // _NW

        def a_v0(i):
            return (wid + i * _NW) * 128

        def a_start_in(i, slot):
            pltpu.async_copy(
                tab_hbm.at[:, pl.ds(a_v0(i), 128)], ain_v.at[slot], isem[slot]
            )

        def a_wait_in(i, slot):
            pltpu.make_async_copy(
                tab_hbm.at[:, pl.ds(a_v0(i), 128)], ain_v.at[slot], isem[slot]
            ).wait()

        def a_dst(i):
            return scr_hbm.at[pl.ds(a_v0(i) // 2, D)]

        def a_transpose(slot):
            row0 = lax.shift_right_logical(iot, 1)
            hvec = jnp.bitwise_and(iot, 1) * D

            @plsc.parallel_loop(0, _L, unroll=2)
            def arot(r):
                rot = jnp.bitwise_and(iot + r, _L - 1)
                for jb2 in range(128 // _L):
                    colvec = iot + jb2 * _L
                    rowvec = row0 + (jb2 * (_L // 2))
                    vals = [plsc.load_gather(ain_v.at[slot],
                                             [rot + (db * _L), colvec])
                            for db in range(D // _L)]
                    for db in range(D // _L):
                        plsc.store_scatter(apk_v.at[slot],
                                           [rowvec, hvec + rot + (db * _L)],
                                           vals[db])

        def a_body(i, slot):
            @pl.when(i + 1 < nc_w)
            def _():
                a_start_in(i + 1, 1 - slot)

            a_wait_in(i, slot)

            @pl.when(i >= 2)
            def _():
                pltpu.make_async_copy(apk_v.at[slot], a_dst(i - 2),
                                      oasem[slot]).wait()

            a_transpose(slot)
            pltpu.async_copy(apk_v.at[slot], a_dst(i), oasem[slot])

        a_start_in(0, 0)

        def a_pair(p, carry):
            a_body(2 * p, 0)
            a_body(2 * p + 1, 1)
            return carry

        lax.fori_loop(0, nc_w // 2, a_pair, 0)

        @pl.when(lax.rem(nc_w, 2) == 1)
        def _():
            a_body(nc_w - 1, 0)

        @pl.when(lax.rem(nc_w, 2) == 1)
        def _():
            pltpu.make_async_copy(apk_v.at[1], a_dst(nc_w - 2), oasem[1]).wait()
            pltpu.make_async_copy(apk_v.at[0], a_dst(nc_w - 1), oasem[0]).wait()

        @pl.when(lax.rem(nc_w, 2) == 0)
        def _():
            pltpu.make_async_copy(apk_v.at[0], a_dst(nc_w - 2), oasem[0]).wait()
            pltpu.make_async_copy(apk_v.at[1], a_dst(nc_w - 1), oasem[1]).wait()

        @pl.when(wid == 0)
        def _():
            pltpu.sync_copy(tail_hbm, g_v.at[0])

            def trot(r, carry):
                rot = jnp.bitwise_and(iot + r, _L - 1)

                def tblk(jb2, carry2):
                    colvec = iot + jb2 * _L
                    rowvec = (jb2 * (_L // 2)) + lax.shift_right_logical(iot, 1)
                    hvec = jnp.bitwise_and(iot, 1) * D

                    def tdb(db, carry3):
                        dvec = rot + (db * _L)
                        vals = plsc.load_gather(g_v.at[0], [dvec, colvec])
                        plsc.store_scatter(apk_v.at[0],
                                           [rowvec, hvec + dvec], vals)
                        return carry3
                    return lax.fori_loop(0, D // _L, tdb, carry2)
                return lax.fori_loop(0, tail_w // _L, tblk, carry)

            lax.fori_loop(0, _L, trot, 0)

            pltpu.sync_copy(apk_v.at[0, pl.ds(0, tail_w // 2)],
                            scr_hbm.at[pl.ds(tail_v0 // 2, tail_w // 2)])

        plsc.subcore_barrier()

        @pl.when(sid == 0)
        def _():
            pl.semaphore_signal(xsem, 1, core_index=1 - cid)
            pl.semaphore_wait(xsem, 1)

        plsc.subcore_barrier()

        b0 = wid * bpw
        pltpu.sync_copy(idx_hbm.at[:, pl.ds(b0, bpw)], idx_v)

        def halve(f, carry):
            for jb in range(bpw // _L):
                v16 = idx_v[f, pl.ds(jb * _L, _L)]
                kidx_v[f, pl.ds(jb * _L, _L)] = lax.shift_right_logical(v16, 1)
            return carry

        lax.fori_loop(0, F, halve, 0)

        def b_src(s):
            f = s // (bpw // _BCH)
            c = lax.rem(s, bpw // _BCH)
            return scr_hbm.at[kidx_v.at[f, pl.ds(c * _BCH, _BCH)]]

        def b_start_gather(s, slot):
            pltpu.async_copy(b_src(s), g_v.at[slot], gsem[slot])

        def b_wait_gather(s, slot):
            pltpu.make_async_copy(b_src(s), g_v.at[slot], gsem[slot]).wait()

        def b_dst(s):
            f = s // (bpw // _BCH)
            c = lax.rem(s, bpw // _BCH)
            return out_hbm.at[f, :, pl.ds(b0 + c * _BCH, _BCH)]

        def b_transpose(s, slot):
            f = s // (bpw // _BCH)
            c = lax.rem(s, bpw // _BCH)

            @plsc.parallel_loop(0, _L, unroll=2)
            def brot(r):
                rot = jnp.bitwise_and(iot + r, _L - 1)
                for jb in range(_BCH // _L):
                    j0 = jb * _L
                    jvec = iot + j0
                    v16 = idx_v[f, pl.ds(c * _BCH + j0, _L)]
                    hv = lax.shift_left(jnp.bitwise_and(v16, 1), 6)
                    vals = [plsc.load_gather(g_v.at[slot],
                                             [jvec, rot + (db * _L) + hv])
                            for db in range(D // _L)]
                    for db in range(D // _L):
                        plsc.store_scatter(t_v.at[slot],
                                           [rot + (db * _L), jvec], vals[db])

        def b_body(s, slot):
            @pl.when(s + 1 < steps)
            def _():
                b_start_gather(s + 1, 1 - slot)

            b_wait_gather(s, slot)

            @pl.when(s >= 2)
            def _():
                pltpu.make_async_copy(t_v.at[slot], b_dst(s - 2),
                                      obsem[slot]).wait()

            b_transpose(s, slot)
            pltpu.async_copy(t_v.at[slot], b_dst(s), obsem[slot])

        b_start_gather(0, 0)

        def b_pair(p, carry):
            b_body(2 * p, 0)
            b_body(2 * p + 1, 1)
            return carry

        lax.fori_loop(0, steps // 2, b_pair, 0)

        pltpu.make_async_copy(t_v.at[0], b_dst(steps - 2), obsem[0]).wait()
        pltpu.make_async_copy(t_v.at[1], b_dst(steps - 1), obsem[1]).wait()

    out_t = run(table_t, idx_t, tail_t)
    return jnp.transpose(out_t, (2, 0, 1))
